# baseline (speedup 1.0000x reference)
import functools
import jax
import jax.numpy as jnp
from jax.experimental import pallas as pl
from jax.experimental.pallas import tpu as pltpu

N_NODES = 10000
HIDDEN = 64


def _mm_kernel(x_ref, w_ref, b_ref, o_ref, *, act):
    y = jnp.dot(x_ref[...], w_ref[...], preferred_element_type=jnp.float32)
    y = y + b_ref[...]
    if act == "relu":
        y = jnp.maximum(y, 0.0)
    o_ref[...] = y


def _dense(x, w, b, act=None):
    n, _ = x.shape
    m = w.shape[1]
    return pl.pallas_call(
        functools.partial(_mm_kernel, act=act),
        out_shape=jax.ShapeDtypeStruct((n, m), jnp.float32),
    )(x, w, b.reshape(1, m))


def _sheaf_layer_xla(h, row, col, rW1, rb1, rW2, rb2, alpha):
    w1a = rW1[:HIDDEN]
    w1b = rW1[HIDDEN:]
    a = _dense(h, w1a, jnp.zeros((rW1.shape[1],), jnp.float32))
    b = _dense(h, w1b, rb1)
    t = jnp.tanh(a[row] + b[col])
    r = jnp.tanh(t @ rW2 + rb2)
    msg = h[row] * r
    aggr = jax.ops.segment_sum(msg, col, num_segments=h.shape[0])
    return h - alpha * (h - aggr)


def kernel(x, edge_index, encW, encb, s1_rW1, s1_rb1, s1_rW2, s1_rb2, s1_alpha,
           s2_rW1, s2_rb1, s2_rW2, s2_rb2, s2_alpha, d1W, d1b, d2W, d2b):
    row = edge_index[0].astype(jnp.int32)
    col = edge_index[1].astype(jnp.int32)
    h = _dense(x, encW, encb, act="relu")
    h = _sheaf_layer_xla(h, row, col, s1_rW1, s1_rb1, s1_rW2, s1_rb2, s1_alpha)
    h = jnp.maximum(h, 0.0)
    h = _sheaf_layer_xla(h, row, col, s2_rW1, s2_rb1, s2_rW2, s2_rb2, s2_alpha)
    h = _dense(h, d1W, d1b, act="relu")
    out = _dense(h, d2W, d2b)
    return out


# speedup vs baseline: 8.9502x; 8.9502x over previous
import dataclasses
import functools
import jax
import jax.numpy as jnp
from jax import lax
from jax.experimental import pallas as pl
from jax.experimental.pallas import tpu as pltpu
from jax.experimental.pallas import tpu_sc as plsc

N_NODES = 10000
N_EDGES = 320000
HIDDEN = 64
PROJ = 32

N_WORKERS = 32
EDGES_PER_WORKER = N_EDGES // N_WORKERS
CHUNK = 125
N_CHUNKS = EDGES_PER_WORKER // CHUNK
ROWS_PER_SUB = N_NODES // 16


def _tanh16(v):
    return 1.0 - 2.0 / (jnp.exp(v + v) + 1.0)


def _sc_edge_body(row_hbm, col_hbm, ha_hbm, b_hbm, w2_hbm, out_hbm,
                  rowv, colv, hav, bv, msgv, w2v, accum):
    core = lax.axis_index("c")
    sub = lax.axis_index("s")
    wid = core * 16 + sub

    pltpu.sync_copy(row_hbm.at[wid], rowv)
    pltpu.sync_copy(col_hbm.at[wid], colv)
    pltpu.sync_copy(w2_hbm, w2v)

    zero16 = jnp.zeros((16,), jnp.float32)

    @pl.loop(0, CHUNK)
    def _zero(e):
        for k in range(HIDDEN // 16):
            msgv[e, pl.ds(k * 16, 16)] = zero16

    @pl.loop(0, ROWS_PER_SUB // CHUNK)
    def _init(k):
        pltpu.sync_copy(msgv, accum.at[pl.ds(sub * ROWS_PER_SUB + k * CHUNK,
                                             CHUNK)])

    plsc.subcore_barrier()

    w2a = w2v[0]
    w2b = w2v[1]
    rb2v = w2v[2]

    @pl.loop(0, N_CHUNKS)
    def _chunk(j):
        pltpu.sync_copy(ha_hbm.at[rowv.at[j]], hav)
        pltpu.sync_copy(b_hbm.at[colv.at[j]], bv)

        @plsc.parallel_loop(0, CHUNK, unroll=5)
        def _edge(e):
            a0 = hav[e, pl.ds(HIDDEN, 16)]
            a1 = hav[e, pl.ds(HIDDEN + 16, 16)]
            b0 = bv[e, pl.ds(0, 16)]
            b1 = bv[e, pl.ds(16, 16)]
            t0 = _tanh16(a0 + b0)
            t1 = _tanh16(a1 + b1)
            m = t0 * w2a + t1 * w2b
            u = jnp.sum(m)
            rv = _tanh16(jnp.full((16,), u, jnp.float32) + rb2v)
            for k in range(HIDDEN // 16):
                msgv[e, pl.ds(k * 16, 16)] = hav[e, pl.ds(k * 16, 16)] * rv

        pltpu.sync_copy(msgv, accum.at[colv.at[j]], add=True)

    plsc.subcore_barrier()
    pltpu.sync_copy(accum.at[pl.ds(sub * ROWS_PER_SUB, ROWS_PER_SUB)],
                    out_hbm.at[core, sub])


def _sc_edge_layer(row3d, col3d, ha, b, w2pack):
    mesh = plsc.VectorSubcoreMesh(core_axis_name="c", subcore_axis_name="s")
    cp = pltpu.CompilerParams()
    if "needs_layout_passes" in pltpu.CompilerParams.__dataclass_fields__:
        cp = dataclasses.replace(cp, needs_layout_passes=False)
    if "use_tc_tiling_on_sc" in pltpu.CompilerParams.__dataclass_fields__:
        cp = dataclasses.replace(cp, use_tc_tiling_on_sc=False)
    f = pl.kernel(
        _sc_edge_body,
        out_type=jax.ShapeDtypeStruct((2, 16, ROWS_PER_SUB, HIDDEN),
                                      jnp.float32),
        mesh=mesh,
        compiler_params=cp,
        scratch_types=[
            pltpu.VMEM((N_CHUNKS, CHUNK), jnp.int32),
            pltpu.VMEM((N_CHUNKS, CHUNK), jnp.int32),
            pltpu.VMEM((CHUNK, HIDDEN + PROJ), jnp.float32),
            pltpu.VMEM((CHUNK, PROJ), jnp.float32),
            pltpu.VMEM((CHUNK, HIDDEN), jnp.float32),
            pltpu.VMEM((4, 16), jnp.float32),
            pltpu.VMEM_SHARED((N_NODES, HIDDEN), jnp.float32),
        ],
    )
    return f(row3d, col3d, ha, b, w2pack).reshape(2, N_NODES, HIDDEN)


def _pre_body(x_ref, encW_ref, encb_ref, w1a_ref, w1b_ref, rb1_ref,
              h_ref, a_ref, b_ref):
    h = jnp.dot(x_ref[...], encW_ref[...], preferred_element_type=jnp.float32)
    h = jnp.maximum(h + encb_ref[...], 0.0)
    h_ref[...] = h
    a_ref[...] = jnp.dot(h, w1a_ref[...], preferred_element_type=jnp.float32)
    b_ref[...] = jnp.dot(h, w1b_ref[...],
                         preferred_element_type=jnp.float32) + rb1_ref[...]


def _mid_body(h_ref, p_ref, alpha_ref, w1a_ref, w1b_ref, rb1_ref,
              h_ref_o, a_ref, b_ref):
    aggr = p_ref[0] + p_ref[1]
    h = h_ref[...]
    alpha = alpha_ref[0, 0]
    h = jnp.maximum(h - alpha * (h - aggr), 0.0)
    h_ref_o[...] = h
    a_ref[...] = jnp.dot(h, w1a_ref[...], preferred_element_type=jnp.float32)
    b_ref[...] = jnp.dot(h, w1b_ref[...],
                         preferred_element_type=jnp.float32) + rb1_ref[...]


def _post_body(h_ref, p_ref, alpha_ref, d1W_ref, d1b_ref, d2W_ref, d2b_ref,
               o_ref):
    aggr = p_ref[0] + p_ref[1]
    h = h_ref[...]
    alpha = alpha_ref[0, 0]
    h = h - alpha * (h - aggr)
    g = jnp.dot(h, d1W_ref[...], preferred_element_type=jnp.float32)
    g = jnp.maximum(g + d1b_ref[...], 0.0)
    o_ref[...] = jnp.dot(g, d2W_ref[...],
                         preferred_element_type=jnp.float32) + d2b_ref[...]


def _f32(*shapes):
    return [jax.ShapeDtypeStruct(s, jnp.float32) for s in shapes]


def kernel(x, edge_index, encW, encb, s1_rW1, s1_rb1, s1_rW2, s1_rb2, s1_alpha,
           s2_rW1, s2_rb1, s2_rW2, s2_rb2, s2_alpha, d1W, d1b, d2W, d2b):
    row3d = edge_index[0].astype(jnp.int32).reshape(N_WORKERS, N_CHUNKS, CHUNK)
    col3d = edge_index[1].astype(jnp.int32).reshape(N_WORKERS, N_CHUNKS, CHUNK)

    def w2pack(rW2, rb2):
        return jnp.concatenate([
            rW2[:16, 0], rW2[16:, 0], jnp.full((16,), rb2[0], jnp.float32),
            jnp.zeros((16,), jnp.float32)]).reshape(4, 16)

    w2p1 = w2pack(s1_rW2, s1_rb2)
    w2p2 = w2pack(s2_rW2, s2_rb2)

    h0, a1, b1 = pl.pallas_call(
        _pre_body,
        out_shape=_f32((N_NODES, HIDDEN), (N_NODES, PROJ), (N_NODES, PROJ)),
    )(x, encW, encb.reshape(1, HIDDEN), s1_rW1[:HIDDEN], s1_rW1[HIDDEN:],
      s1_rb1.reshape(1, PROJ))

    ha1 = jnp.concatenate([h0, a1], axis=1)
    p1 = _sc_edge_layer(row3d, col3d, ha1, b1, w2p1)

    h1, a2, b2 = pl.pallas_call(
        _mid_body,
        out_shape=_f32((N_NODES, HIDDEN), (N_NODES, PROJ), (N_NODES, PROJ)),
    )(h0, p1, s1_alpha.reshape(1, 1), s2_rW1[:HIDDEN], s2_rW1[HIDDEN:],
      s2_rb1.reshape(1, PROJ))

    ha2 = jnp.concatenate([h1, a2], axis=1)
    p2 = _sc_edge_layer(row3d, col3d, ha2, b2, w2p2)

    out = pl.pallas_call(
        _post_body,
        out_shape=jax.ShapeDtypeStruct((N_NODES, 2), jnp.float32),
    )(h1, p2, s2_alpha.reshape(1, 1), d1W, d1b.reshape(1, PROJ), d2W,
      d2b.reshape(1, 2))
    return out


# speedup vs baseline: 15.9839x; 1.7859x over previous
import dataclasses
import functools
import jax
import jax.numpy as jnp
from jax import lax
from jax.experimental import pallas as pl
from jax.experimental.pallas import tpu as pltpu
from jax.experimental.pallas import tpu_sc as plsc

N_NODES = 10000
N_EDGES = 320000
HIDDEN = 64
PROJ = 32

N_WORKERS = 32
EDGES_PER_WORKER = N_EDGES // N_WORKERS
CHUNK = 125
N_CHUNKS = EDGES_PER_WORKER // CHUNK
ROWS_PER_SUB = N_NODES // 16


def _tanh16(v):
    return 1.0 - 2.0 / (jnp.exp(v + v) + 1.0)


def _sc_edge_body(row_hbm, col_hbm, ha_hbm, b_hbm, w2_hbm, out_hbm,
                  rowv, colv, hav0, bv0, hav1, bv1, msgv, w2v, accum,
                  sg0, sg1):
    core = lax.axis_index("c")
    sub = lax.axis_index("s")
    wid = core * 16 + sub

    pltpu.sync_copy(row_hbm.at[wid], rowv)
    pltpu.sync_copy(col_hbm.at[wid], colv)
    pltpu.sync_copy(w2_hbm, w2v)

    zero16 = jnp.zeros((16,), jnp.float32)

    @pl.loop(0, CHUNK)
    def _zero(e):
        for k in range(HIDDEN // 16):
            msgv[e, pl.ds(k * 16, 16)] = zero16

    @pl.loop(0, ROWS_PER_SUB // CHUNK)
    def _init(k):
        pltpu.sync_copy(msgv, accum.at[pl.ds(sub * ROWS_PER_SUB + k * CHUNK,
                                             CHUNK)])

    plsc.subcore_barrier()

    w2a = w2v[0]
    w2b = w2v[1]
    rb2v = w2v[2]

    def start_gathers(j, hav, bv, sem):
        pltpu.async_copy(ha_hbm.at[rowv.at[j]], hav, sem)
        pltpu.async_copy(b_hbm.at[colv.at[j]], bv, sem)

    def wait_gathers(hav, bv, sem):
        pltpu.make_async_copy(ha_hbm.at[pl.ds(0, CHUNK)], hav, sem).wait()
        pltpu.make_async_copy(b_hbm.at[pl.ds(0, CHUNK)], bv, sem).wait()

    def process(j, hav, bv):
        @plsc.parallel_loop(0, CHUNK, unroll=5)
        def _edge(e):
            a0 = hav[e, pl.ds(HIDDEN, 16)]
            a1 = hav[e, pl.ds(HIDDEN + 16, 16)]
            b0 = bv[e, pl.ds(0, 16)]
            b1 = bv[e, pl.ds(16, 16)]
            t0 = _tanh16(a0 + b0)
            t1 = _tanh16(a1 + b1)
            m = t0 * w2a + t1 * w2b
            u = jnp.sum(m)
            rv = _tanh16(jnp.full((16,), u, jnp.float32) + rb2v)
            for k in range(HIDDEN // 16):
                msgv[e, pl.ds(k * 16, 16)] = hav[e, pl.ds(k * 16, 16)] * rv

        pltpu.sync_copy(msgv, accum.at[colv.at[j]], add=True)

    start_gathers(0, hav0, bv0, sg0)

    @pl.loop(0, N_CHUNKS - 2, step=2)
    def _chunk(j):
        start_gathers(j + 1, hav1, bv1, sg1)
        wait_gathers(hav0, bv0, sg0)
        process(j, hav0, bv0)
        start_gathers(j + 2, hav0, bv0, sg0)
        wait_gathers(hav1, bv1, sg1)
        process(j + 1, hav1, bv1)

    start_gathers(N_CHUNKS - 1, hav1, bv1, sg1)
    wait_gathers(hav0, bv0, sg0)
    process(N_CHUNKS - 2, hav0, bv0)
    wait_gathers(hav1, bv1, sg1)
    process(N_CHUNKS - 1, hav1, bv1)

    plsc.subcore_barrier()
    pltpu.sync_copy(accum.at[pl.ds(sub * ROWS_PER_SUB, ROWS_PER_SUB)],
                    out_hbm.at[core, sub])


def _sc_edge_layer(row3d, col3d, ha, b, w2pack):
    mesh = plsc.VectorSubcoreMesh(core_axis_name="c", subcore_axis_name="s")
    cp = pltpu.CompilerParams()
    if "needs_layout_passes" in pltpu.CompilerParams.__dataclass_fields__:
        cp = dataclasses.replace(cp, needs_layout_passes=False)
    if "use_tc_tiling_on_sc" in pltpu.CompilerParams.__dataclass_fields__:
        cp = dataclasses.replace(cp, use_tc_tiling_on_sc=False)
    f = pl.kernel(
        _sc_edge_body,
        out_type=jax.ShapeDtypeStruct((2, 16, ROWS_PER_SUB, HIDDEN),
                                      jnp.float32),
        mesh=mesh,
        compiler_params=cp,
        scratch_types=[
            pltpu.VMEM((N_CHUNKS, CHUNK), jnp.int32),
            pltpu.VMEM((N_CHUNKS, CHUNK), jnp.int32),
            pltpu.VMEM((CHUNK, HIDDEN + PROJ), jnp.float32),
            pltpu.VMEM((CHUNK, PROJ), jnp.float32),
            pltpu.VMEM((CHUNK, HIDDEN + PROJ), jnp.float32),
            pltpu.VMEM((CHUNK, PROJ), jnp.float32),
            pltpu.VMEM((CHUNK, HIDDEN), jnp.float32),
            pltpu.VMEM((4, 16), jnp.float32),
            pltpu.VMEM_SHARED((N_NODES, HIDDEN), jnp.float32),
            pltpu.SemaphoreType.DMA,
            pltpu.SemaphoreType.DMA,
        ],
    )
    return f(row3d, col3d, ha, b, w2pack).reshape(2, N_NODES, HIDDEN)


def _pre_body(x_ref, encW_ref, encb_ref, w1a_ref, w1b_ref, rb1_ref,
              h_ref, a_ref, b_ref):
    h = jnp.dot(x_ref[...], encW_ref[...], preferred_element_type=jnp.float32)
    h = jnp.maximum(h + encb_ref[...], 0.0)
    h_ref[...] = h
    a_ref[...] = jnp.dot(h, w1a_ref[...], preferred_element_type=jnp.float32)
    b_ref[...] = jnp.dot(h, w1b_ref[...],
                         preferred_element_type=jnp.float32) + rb1_ref[...]


def _mid_body(h_ref, p_ref, alpha_ref, w1a_ref, w1b_ref, rb1_ref,
              h_ref_o, a_ref, b_ref):
    aggr = p_ref[0] + p_ref[1]
    h = h_ref[...]
    alpha = alpha_ref[0, 0]
    h = jnp.maximum(h - alpha * (h - aggr), 0.0)
    h_ref_o[...] = h
    a_ref[...] = jnp.dot(h, w1a_ref[...], preferred_element_type=jnp.float32)
    b_ref[...] = jnp.dot(h, w1b_ref[...],
                         preferred_element_type=jnp.float32) + rb1_ref[...]


def _post_body(h_ref, p_ref, alpha_ref, d1W_ref, d1b_ref, d2W_ref, d2b_ref,
               o_ref):
    aggr = p_ref[0] + p_ref[1]
    h = h_ref[...]
    alpha = alpha_ref[0, 0]
    h = h - alpha * (h - aggr)
    g = jnp.dot(h, d1W_ref[...], preferred_element_type=jnp.float32)
    g = jnp.maximum(g + d1b_ref[...], 0.0)
    o_ref[...] = jnp.dot(g, d2W_ref[...],
                         preferred_element_type=jnp.float32) + d2b_ref[...]


def _f32(*shapes):
    return [jax.ShapeDtypeStruct(s, jnp.float32) for s in shapes]


def kernel(x, edge_index, encW, encb, s1_rW1, s1_rb1, s1_rW2, s1_rb2, s1_alpha,
           s2_rW1, s2_rb1, s2_rW2, s2_rb2, s2_alpha, d1W, d1b, d2W, d2b):
    row3d = edge_index[0].astype(jnp.int32).reshape(N_WORKERS, N_CHUNKS, CHUNK)
    col3d = edge_index[1].astype(jnp.int32).reshape(N_WORKERS, N_CHUNKS, CHUNK)

    def w2pack(rW2, rb2):
        return jnp.concatenate([
            rW2[:16, 0], rW2[16:, 0], jnp.full((16,), rb2[0], jnp.float32),
            jnp.zeros((16,), jnp.float32)]).reshape(4, 16)

    w2p1 = w2pack(s1_rW2, s1_rb2)
    w2p2 = w2pack(s2_rW2, s2_rb2)

    h0, a1, b1 = pl.pallas_call(
        _pre_body,
        out_shape=_f32((N_NODES, HIDDEN), (N_NODES, PROJ), (N_NODES, PROJ)),
    )(x, encW, encb.reshape(1, HIDDEN), s1_rW1[:HIDDEN], s1_rW1[HIDDEN:],
      s1_rb1.reshape(1, PROJ))

    ha1 = jnp.concatenate([h0, a1], axis=1)
    p1 = _sc_edge_layer(row3d, col3d, ha1, b1, w2p1)

    h1, a2, b2 = pl.pallas_call(
        _mid_body,
        out_shape=_f32((N_NODES, HIDDEN), (N_NODES, PROJ), (N_NODES, PROJ)),
    )(h0, p1, s1_alpha.reshape(1, 1), s2_rW1[:HIDDEN], s2_rW1[HIDDEN:],
      s2_rb1.reshape(1, PROJ))

    ha2 = jnp.concatenate([h1, a2], axis=1)
    p2 = _sc_edge_layer(row3d, col3d, ha2, b2, w2p2)

    out = pl.pallas_call(
        _post_body,
        out_shape=jax.ShapeDtypeStruct((N_NODES, 2), jnp.float32),
    )(h1, p2, s2_alpha.reshape(1, 1), d1W, d1b.reshape(1, PROJ), d2W,
      d2b.reshape(1, 2))
    return out
